# baseline (speedup 1.0000x reference)
import functools

import jax
import jax.numpy as jnp
from jax import lax
from jax.experimental import pallas as pl
from jax.experimental.pallas import tpu as pltpu
from jax.experimental.pallas import tpu_sc as plsc

N = 1_000_000
M = 500_000
NC = 2
NS = 16
NW = NC * NS
ROWS_PER_TILE = N // NW
LOCAL_PAD = 32_000
CHUNK = 10_000
NCHUNK = M // CHUNK
VPC = CHUNK // 16
L = 16

_mesh = plsc.VectorSubcoreMesh(core_axis_name="c", subcore_axis_name="s")


@functools.partial(
    pl.kernel,
    mesh=_mesh,
    out_type=jax.ShapeDtypeStruct((NW, ROWS_PER_TILE), jnp.int32),
    scratch_types=[
        pltpu.VMEM((CHUNK,), jnp.int32),
        pltpu.VMEM((CHUNK,), jnp.int32),
        pltpu.VMEM((LOCAL_PAD,), jnp.int32),
        pltpu.SemaphoreType.DMA,
        pltpu.SemaphoreType.DMA,
    ],
    compiler_params=pltpu.CompilerParams(
        needs_layout_passes=False, use_tc_tiling_on_sc=False),
)
def _scatter_arange(idx_hbm, out_hbm, buf_a, buf_b, local, sem_a, sem_b):
    wid = lax.axis_index("c") * NS + lax.axis_index("s")
    base = (wid * ROWS_PER_TILE).astype(jnp.int32)
    iota = lax.iota(jnp.int32, L)
    zeros = jnp.zeros((L,), jnp.int32)
    limit = jnp.uint32(ROWS_PER_TILE)

    pltpu.async_copy(idx_hbm.at[pl.ds(0, CHUNK)], buf_a, sem_a)

    @plsc.parallel_loop(0, LOCAL_PAD // L, 1, unroll=8)
    def _zero_body(j):
        local[pl.ds(j * L, L)] = zeros

    def _process(buf, chunk_base):
        @plsc.parallel_loop(0, VPC, 1, unroll=25)
        def _vec_body(v):
            ivec = buf[pl.ds(v * L, L)]
            val = (chunk_base + v * L) + iota
            loc = ivec - base
            m = plsc.bitcast(loc, jnp.uint32) < limit
            plsc.store_scatter(local, [loc], val, mask=m)

    def _pair_body(c, _):
        c0 = 2 * c
        pltpu.make_async_copy(idx_hbm.at[pl.ds(0, CHUNK)], buf_a, sem_a).wait()
        pltpu.async_copy(
            idx_hbm.at[pl.ds((c0 + 1) * CHUNK, CHUNK)], buf_b, sem_b)
        _process(buf_a, (c0 * CHUNK).astype(jnp.int32))
        nxt = jnp.minimum(c0 + 2, NCHUNK - 1)
        pltpu.make_async_copy(idx_hbm.at[pl.ds(0, CHUNK)], buf_b, sem_b).wait()
        pltpu.async_copy(idx_hbm.at[pl.ds(nxt * CHUNK, CHUNK)], buf_a, sem_a)
        _process(buf_b, ((c0 + 1) * CHUNK).astype(jnp.int32))
        return 0

    lax.fori_loop(0, NCHUNK // 2, _pair_body, 0)
    pltpu.make_async_copy(idx_hbm.at[pl.ds(0, CHUNK)], buf_a, sem_a).wait()
    pltpu.sync_copy(local.at[pl.ds(0, ROWS_PER_TILE)], out_hbm.at[wid])


def kernel(X, row_missing_idx):
    del X
    return _scatter_arange(row_missing_idx).reshape(-1)


# speedup vs baseline: 1.6527x; 1.6527x over previous
import functools

import jax
import jax.numpy as jnp
from jax import lax
from jax.experimental import pallas as pl
from jax.experimental.pallas import tpu as pltpu
from jax.experimental.pallas import tpu_sc as plsc

N = 1_000_000
M = 500_000
NC = 2
NS = 16
NW = NC * NS
ROWS_PER_TILE = N // NW
LOCAL_PAD = 32_000
CHUNK = 10_000
NCHUNK = M // CHUNK
VPC = CHUNK // 16
L = 16
STG = 31_248
STG_TAIL = M - NS * STG

_mesh = plsc.VectorSubcoreMesh(core_axis_name="c", subcore_axis_name="s")


@functools.partial(
    pl.kernel,
    mesh=_mesh,
    out_type=jax.ShapeDtypeStruct((NW, ROWS_PER_TILE), jnp.int32),
    scratch_types=[
        pltpu.VMEM((CHUNK,), jnp.int32),
        pltpu.VMEM((CHUNK,), jnp.int32),
        pltpu.VMEM((LOCAL_PAD,), jnp.int32),
        pltpu.VMEM_SHARED((M,), jnp.int32),
        pltpu.SemaphoreType.DMA,
        pltpu.SemaphoreType.DMA,
    ],
    compiler_params=pltpu.CompilerParams(
        needs_layout_passes=False, use_tc_tiling_on_sc=False),
)
def _scatter_arange(idx_hbm, out_hbm, buf_a, buf_b, local, shared,
                    sem_a, sem_b):
    wid = lax.axis_index("c") * NS + lax.axis_index("s")
    sid = lax.axis_index("s")
    base = (wid * ROWS_PER_TILE).astype(jnp.int32)
    iota = lax.iota(jnp.int32, L)
    zeros = jnp.zeros((L,), jnp.int32)
    limit = jnp.uint32(ROWS_PER_TILE)

    pltpu.async_copy(idx_hbm.at[pl.ds(sid * STG, STG)],
                     shared.at[pl.ds(sid * STG, STG)], sem_a)

    @pl.when(sid == 0)
    def _tail():
        pltpu.async_copy(idx_hbm.at[pl.ds(NS * STG, STG_TAIL)],
                         shared.at[pl.ds(NS * STG, STG_TAIL)], sem_b)

    @plsc.parallel_loop(0, LOCAL_PAD // L, 1, unroll=8)
    def _zero_body(j):
        local[pl.ds(j * L, L)] = zeros

    pltpu.make_async_copy(idx_hbm.at[pl.ds(0, STG)],
                          shared.at[pl.ds(0, STG)], sem_a).wait()

    @pl.when(sid == 0)
    def _tail_wait():
        pltpu.make_async_copy(idx_hbm.at[pl.ds(0, STG_TAIL)],
                              shared.at[pl.ds(0, STG_TAIL)], sem_b).wait()

    plsc.subcore_barrier()

    def _process(buf, chunk_base):
        @plsc.parallel_loop(0, VPC, 1, unroll=25)
        def _vec_body(v):
            ivec = buf[pl.ds(v * L, L)]
            val = (chunk_base + v * L) + iota
            loc = ivec - base
            m = plsc.bitcast(loc, jnp.uint32) < limit
            plsc.store_scatter(local, [loc], val, mask=m)

    pltpu.async_copy(shared.at[pl.ds(0, CHUNK)], buf_a, sem_a)

    def _pair_body(c, _):
        c0 = 2 * c
        pltpu.make_async_copy(shared.at[pl.ds(0, CHUNK)], buf_a, sem_a).wait()
        pltpu.async_copy(
            shared.at[pl.ds((c0 + 1) * CHUNK, CHUNK)], buf_b, sem_b)
        _process(buf_a, (c0 * CHUNK).astype(jnp.int32))
        nxt = jnp.minimum(c0 + 2, NCHUNK - 1)
        pltpu.make_async_copy(shared.at[pl.ds(0, CHUNK)], buf_b, sem_b).wait()
        pltpu.async_copy(shared.at[pl.ds(nxt * CHUNK, CHUNK)], buf_a, sem_a)
        _process(buf_b, ((c0 + 1) * CHUNK).astype(jnp.int32))
        return 0

    lax.fori_loop(0, NCHUNK // 2, _pair_body, 0)
    pltpu.make_async_copy(shared.at[pl.ds(0, CHUNK)], buf_a, sem_a).wait()
    pltpu.sync_copy(local.at[pl.ds(0, ROWS_PER_TILE)], out_hbm.at[wid])


def kernel(X, row_missing_idx):
    del X
    return _scatter_arange(row_missing_idx).reshape(-1)


# speedup vs baseline: 1.6534x; 1.0004x over previous
import functools

import jax
import jax.numpy as jnp
from jax import lax
from jax.experimental import pallas as pl
from jax.experimental.pallas import tpu as pltpu
from jax.experimental.pallas import tpu_sc as plsc

N = 1_000_000
M = 500_000
NC = 2
NS = 16
NW = NC * NS
ROWS_PER_TILE = N // NW
LOCAL_PAD = 32_000
CHUNK = 10_000
NCHUNK = M // CHUNK
VPC = CHUNK // 16
L = 16
STG = 31_248
STG_TAIL = M - NS * STG

_mesh = plsc.VectorSubcoreMesh(core_axis_name="c", subcore_axis_name="s")


@functools.partial(
    pl.kernel,
    mesh=_mesh,
    out_type=jax.ShapeDtypeStruct((NW, ROWS_PER_TILE), jnp.int32),
    scratch_types=[
        pltpu.VMEM((CHUNK,), jnp.int32),
        pltpu.VMEM((CHUNK,), jnp.int32),
        pltpu.VMEM((LOCAL_PAD,), jnp.int32),
        pltpu.VMEM_SHARED((M,), jnp.int32),
        pltpu.SemaphoreType.DMA,
        pltpu.SemaphoreType.DMA,
    ],
    compiler_params=pltpu.CompilerParams(
        needs_layout_passes=False, use_tc_tiling_on_sc=False,
        disable_bounds_checks=True, skip_device_barrier=True),
)
def _scatter_arange(idx_hbm, out_hbm, buf_a, buf_b, local, shared,
                    sem_a, sem_b):
    wid = lax.axis_index("c") * NS + lax.axis_index("s")
    sid = lax.axis_index("s")
    base = (wid * ROWS_PER_TILE).astype(jnp.int32)
    iota = lax.iota(jnp.int32, L)
    zeros = jnp.zeros((L,), jnp.int32)
    limit = jnp.uint32(ROWS_PER_TILE)

    pltpu.async_copy(idx_hbm.at[pl.ds(sid * STG, STG)],
                     shared.at[pl.ds(sid * STG, STG)], sem_a)

    @pl.when(sid == 0)
    def _tail():
        pltpu.async_copy(idx_hbm.at[pl.ds(NS * STG, STG_TAIL)],
                         shared.at[pl.ds(NS * STG, STG_TAIL)], sem_b)

    @plsc.parallel_loop(0, LOCAL_PAD // L, 1, unroll=8)
    def _zero_body(j):
        local[pl.ds(j * L, L)] = zeros

    pltpu.make_async_copy(idx_hbm.at[pl.ds(0, STG)],
                          shared.at[pl.ds(0, STG)], sem_a).wait()

    @pl.when(sid == 0)
    def _tail_wait():
        pltpu.make_async_copy(idx_hbm.at[pl.ds(0, STG_TAIL)],
                              shared.at[pl.ds(0, STG_TAIL)], sem_b).wait()

    plsc.subcore_barrier()

    def _process(buf, chunk_base):
        @plsc.parallel_loop(0, VPC, 1, unroll=25)
        def _vec_body(v):
            ivec = buf[pl.ds(v * L, L)]
            val = (chunk_base + v * L) + iota
            loc = ivec - base
            m = plsc.bitcast(loc, jnp.uint32) < limit
            plsc.store_scatter(local, [loc], val, mask=m)

    pltpu.async_copy(shared.at[pl.ds(0, CHUNK)], buf_a, sem_a)

    def _pair_body(c, _):
        c0 = 2 * c
        pltpu.make_async_copy(shared.at[pl.ds(0, CHUNK)], buf_a, sem_a).wait()
        pltpu.async_copy(
            shared.at[pl.ds((c0 + 1) * CHUNK, CHUNK)], buf_b, sem_b)
        _process(buf_a, (c0 * CHUNK).astype(jnp.int32))
        nxt = jnp.minimum(c0 + 2, NCHUNK - 1)
        pltpu.make_async_copy(shared.at[pl.ds(0, CHUNK)], buf_b, sem_b).wait()
        pltpu.async_copy(shared.at[pl.ds(nxt * CHUNK, CHUNK)], buf_a, sem_a)
        _process(buf_b, ((c0 + 1) * CHUNK).astype(jnp.int32))
        return 0

    lax.fori_loop(0, NCHUNK // 2, _pair_body, 0)
    pltpu.make_async_copy(shared.at[pl.ds(0, CHUNK)], buf_a, sem_a).wait()
    pltpu.sync_copy(local.at[pl.ds(0, ROWS_PER_TILE)], out_hbm.at[wid])


def kernel(X, row_missing_idx):
    del X
    return _scatter_arange(row_missing_idx).reshape(-1)


# speedup vs baseline: 1.7081x; 1.0331x over previous
import functools

import jax
import jax.numpy as jnp
from jax import lax
from jax.experimental import pallas as pl
from jax.experimental.pallas import tpu as pltpu
from jax.experimental.pallas import tpu_sc as plsc

N = 1_000_000
M = 500_000
NC = 2
NS = 16
NW = NC * NS
ROWS_PER_TILE = N // NW
LOCAL_PAD = 32_000
CHUNK = 20_000
NCHUNK = M // CHUNK
VPC = CHUNK // 16
L = 16
STG = 31_248
STG_TAIL = M - NS * STG

_mesh = plsc.VectorSubcoreMesh(core_axis_name="c", subcore_axis_name="s")


@functools.partial(
    pl.kernel,
    mesh=_mesh,
    out_type=jax.ShapeDtypeStruct((NW, ROWS_PER_TILE), jnp.int32),
    scratch_types=[
        pltpu.VMEM((CHUNK,), jnp.int32),
        pltpu.VMEM((CHUNK,), jnp.int32),
        pltpu.VMEM((LOCAL_PAD,), jnp.int32),
        pltpu.VMEM_SHARED((M,), jnp.int32),
        pltpu.SemaphoreType.DMA,
        pltpu.SemaphoreType.DMA,
    ],
    compiler_params=pltpu.CompilerParams(
        needs_layout_passes=False, use_tc_tiling_on_sc=False),
)
def _scatter_arange(idx_hbm, out_hbm, buf_a, buf_b, local, shared,
                    sem_a, sem_b):
    wid = lax.axis_index("c") * NS + lax.axis_index("s")
    sid = lax.axis_index("s")
    base = (wid * ROWS_PER_TILE).astype(jnp.int32)
    iota = lax.iota(jnp.int32, L)
    zeros = jnp.zeros((L,), jnp.int32)
    limit = jnp.uint32(ROWS_PER_TILE)

    pltpu.async_copy(idx_hbm.at[pl.ds(sid * STG, STG)],
                     shared.at[pl.ds(sid * STG, STG)], sem_a)

    @pl.when(sid == 0)
    def _tail():
        pltpu.async_copy(idx_hbm.at[pl.ds(NS * STG, STG_TAIL)],
                         shared.at[pl.ds(NS * STG, STG_TAIL)], sem_b)

    @plsc.parallel_loop(0, LOCAL_PAD // L, 1, unroll=8)
    def _zero_body(j):
        local[pl.ds(j * L, L)] = zeros

    pltpu.make_async_copy(idx_hbm.at[pl.ds(0, STG)],
                          shared.at[pl.ds(0, STG)], sem_a).wait()

    @pl.when(sid == 0)
    def _tail_wait():
        pltpu.make_async_copy(idx_hbm.at[pl.ds(0, STG_TAIL)],
                              shared.at[pl.ds(0, STG_TAIL)], sem_b).wait()

    plsc.subcore_barrier()

    def _process(buf, chunk_base):
        @plsc.parallel_loop(0, VPC, 1, unroll=25)
        def _vec_body(v):
            ivec = buf[pl.ds(v * L, L)]
            val = (chunk_base + v * L) + iota
            loc = ivec - base
            m = plsc.bitcast(loc, jnp.uint32) < limit
            plsc.store_scatter(local, [loc], val, mask=m)

    pltpu.async_copy(shared.at[pl.ds(0, CHUNK)], buf_a, sem_a)

    def _pair_body(c, _):
        c0 = 2 * c
        pltpu.make_async_copy(shared.at[pl.ds(0, CHUNK)], buf_a, sem_a).wait()
        pltpu.async_copy(
            shared.at[pl.ds((c0 + 1) * CHUNK, CHUNK)], buf_b, sem_b)
        _process(buf_a, (c0 * CHUNK).astype(jnp.int32))
        pltpu.make_async_copy(shared.at[pl.ds(0, CHUNK)], buf_b, sem_b).wait()
        pltpu.async_copy(
            shared.at[pl.ds((c0 + 2) * CHUNK, CHUNK)], buf_a, sem_a)
        _process(buf_b, ((c0 + 1) * CHUNK).astype(jnp.int32))
        return 0

    lax.fori_loop(0, NCHUNK // 2, _pair_body, 0)
    pltpu.make_async_copy(shared.at[pl.ds(0, CHUNK)], buf_a, sem_a).wait()
    _process(buf_a, jnp.int32((NCHUNK - 1) * CHUNK))
    pltpu.sync_copy(local.at[pl.ds(0, ROWS_PER_TILE)], out_hbm.at[wid])


def kernel(X, row_missing_idx):
    del X
    return _scatter_arange(row_missing_idx).reshape(-1)


# speedup vs baseline: 1.7920x; 1.0491x over previous
import functools

import jax
import jax.numpy as jnp
from jax import lax
from jax.experimental import pallas as pl
from jax.experimental.pallas import tpu as pltpu
from jax.experimental.pallas import tpu_sc as plsc

N = 1_000_000
M = 500_000
NC = 2
NS = 16
NW = NC * NS
NPAIR = NW // 2
PAIR_ROWS = N // NPAIR
SPLIT = 32_768
HI_ROWS = PAIR_ROWS - SPLIT
LOCAL_PAD = 62_528
GIVE_HI = LOCAL_PAD - SPLIT
CHUNK = 10_000
NCHALF = (M // 2) // CHUNK
VPC = CHUNK // 16
L = 16
STG = 31_248
STG_TAIL = M - NS * STG
MRG_A = 8_192
MRG_B = 4_960

_mesh = plsc.VectorSubcoreMesh(core_axis_name="c", subcore_axis_name="s")


@functools.partial(
    pl.kernel,
    mesh=_mesh,
    out_type=jax.ShapeDtypeStruct((NPAIR, PAIR_ROWS), jnp.int32),
    scratch_types=[
        pltpu.VMEM((CHUNK,), jnp.int32),
        pltpu.VMEM((CHUNK,), jnp.int32),
        pltpu.VMEM((LOCAL_PAD,), jnp.int32),
        pltpu.VMEM_SHARED((NS * SPLIT,), jnp.int32),
        pltpu.SemaphoreType.DMA,
        pltpu.SemaphoreType.DMA,
    ],
    compiler_params=pltpu.CompilerParams(
        needs_layout_passes=False, use_tc_tiling_on_sc=False),
)
def _scatter_arange(idx_hbm, out_hbm, buf_a, buf_b, local, shared,
                    sem_a, sem_b):
    mrg = shared
    sid = lax.axis_index("s")
    wid = lax.axis_index("c") * NS + sid
    half = wid % 2
    prow = wid // 2
    pbase = (prow * PAIR_ROWS).astype(jnp.int32)
    ibase = half * (M // 2)
    iota = lax.iota(jnp.int32, L)
    zeros = jnp.zeros((L,), jnp.int32)
    limit = jnp.uint32(PAIR_ROWS)

    pltpu.async_copy(idx_hbm.at[pl.ds(sid * STG, STG)],
                     shared.at[pl.ds(sid * STG, STG)], sem_a)

    @pl.when(sid == 0)
    def _tail():
        pltpu.async_copy(idx_hbm.at[pl.ds(NS * STG, STG_TAIL)],
                         shared.at[pl.ds(NS * STG, STG_TAIL)], sem_b)

    @plsc.parallel_loop(0, LOCAL_PAD // L, 1, unroll=4)
    def _zero_body(j):
        local[pl.ds(j * L, L)] = zeros

    pltpu.make_async_copy(idx_hbm.at[pl.ds(0, STG)],
                          shared.at[pl.ds(0, STG)], sem_a).wait()

    @pl.when(sid == 0)
    def _tail_wait():
        pltpu.make_async_copy(idx_hbm.at[pl.ds(0, STG_TAIL)],
                              shared.at[pl.ds(0, STG_TAIL)], sem_b).wait()

    plsc.subcore_barrier()

    def _process(buf, chunk_base):
        @plsc.parallel_loop(0, VPC, 1, unroll=25)
        def _vec_body(v):
            ivec = buf[pl.ds(v * L, L)]
            val = (chunk_base + v * L) + iota
            loc = ivec - pbase
            m = plsc.bitcast(loc, jnp.uint32) < limit
            plsc.store_scatter(local, [loc], val, mask=m)

    pltpu.async_copy(shared.at[pl.ds(ibase, CHUNK)], buf_a, sem_a)

    def _pair_body(c, _):
        c0 = 2 * c
        pltpu.make_async_copy(shared.at[pl.ds(0, CHUNK)], buf_a, sem_a).wait()
        pltpu.async_copy(
            shared.at[pl.ds(ibase + (c0 + 1) * CHUNK, CHUNK)], buf_b, sem_b)
        _process(buf_a, (ibase + c0 * CHUNK).astype(jnp.int32))
        pltpu.make_async_copy(shared.at[pl.ds(0, CHUNK)], buf_b, sem_b).wait()
        pltpu.async_copy(
            shared.at[pl.ds(ibase + (c0 + 2) * CHUNK, CHUNK)], buf_a, sem_a)
        _process(buf_b, (ibase + (c0 + 1) * CHUNK).astype(jnp.int32))
        return 0

    lax.fori_loop(0, NCHALF // 2, _pair_body, 0)
    pltpu.make_async_copy(shared.at[pl.ds(0, CHUNK)], buf_a, sem_a).wait()
    _process(buf_a, (ibase + (NCHALF - 1) * CHUNK).astype(jnp.int32))

    plsc.subcore_barrier()

    @pl.when(half == 0)
    def _give_hi():
        pltpu.sync_copy(local.at[pl.ds(SPLIT, GIVE_HI)],
                        mrg.at[pl.ds(sid * SPLIT, GIVE_HI)])

    @pl.when(half == 1)
    def _give_lo():
        pltpu.sync_copy(local.at[pl.ds(0, SPLIT)],
                        mrg.at[pl.ds(sid * SPLIT, SPLIT)])

    plsc.subcore_barrier()

    @pl.when(half == 0)
    def _merge_lo():
        for k in range(SPLIT // MRG_A):
            pltpu.sync_copy(
                mrg.at[pl.ds((sid + 1) * SPLIT + k * MRG_A, MRG_A)],
                buf_a.at[pl.ds(0, MRG_A)])

            @plsc.parallel_loop(0, MRG_A // L, 1, unroll=8)
            def _mx(j):
                o = k * MRG_A + j * L
                local[pl.ds(o, L)] = jnp.maximum(
                    local[pl.ds(o, L)], buf_a[pl.ds(j * L, L)])

        pltpu.sync_copy(local.at[pl.ds(0, SPLIT)],
                        out_hbm.at[prow, pl.ds(0, SPLIT)])

    @pl.when(half == 1)
    def _merge_hi():
        for k in range(GIVE_HI // MRG_B):
            pltpu.sync_copy(
                mrg.at[pl.ds((sid - 1) * SPLIT + k * MRG_B, MRG_B)],
                buf_a.at[pl.ds(0, MRG_B)])

            @plsc.parallel_loop(0, MRG_B // L, 1, unroll=5)
            def _mx(j):
                o = SPLIT + k * MRG_B + j * L
                local[pl.ds(o, L)] = jnp.maximum(
                    local[pl.ds(o, L)], buf_a[pl.ds(j * L, L)])

        pltpu.sync_copy(local.at[pl.ds(SPLIT, HI_ROWS)],
                        out_hbm.at[prow, pl.ds(SPLIT, HI_ROWS)])


def kernel(X, row_missing_idx):
    del X
    return _scatter_arange(row_missing_idx).reshape(-1)


# speedup vs baseline: 1.8406x; 1.0272x over previous
import functools

import jax
import jax.numpy as jnp
from jax import lax
from jax.experimental import pallas as pl
from jax.experimental.pallas import tpu as pltpu
from jax.experimental.pallas import tpu_sc as plsc

N = 1_000_000
M = 500_000
NC = 2
NS = 16
NW = NC * NS
NPAIR = NW // 2
PAIR_ROWS = N // NPAIR
SPLIT = 32_768
HI_ROWS = PAIR_ROWS - SPLIT
LOCAL_PAD = 62_528
GIVE_HI = LOCAL_PAD - SPLIT
CHUNK = 10_000
NCHALF = (M // 2) // CHUNK
VPC = CHUNK // 16
L = 16
STG = 31_248
STG_TAIL = M - NS * STG
MRG_A = 8_192
MRG_B = 4_960

_mesh = plsc.VectorSubcoreMesh(core_axis_name="c", subcore_axis_name="s")


@functools.partial(
    pl.kernel,
    mesh=_mesh,
    out_type=jax.ShapeDtypeStruct((NPAIR, PAIR_ROWS), jnp.int32),
    scratch_types=[
        pltpu.VMEM((CHUNK,), jnp.int32),
        pltpu.VMEM((CHUNK,), jnp.int32),
        pltpu.VMEM((LOCAL_PAD,), jnp.int32),
        pltpu.VMEM_SHARED((NS * SPLIT,), jnp.int32),
        pltpu.SemaphoreType.DMA,
        pltpu.SemaphoreType.DMA,
    ],
    compiler_params=pltpu.CompilerParams(
        needs_layout_passes=False, use_tc_tiling_on_sc=False),
)
def _scatter_arange(idx_hbm, out_hbm, buf_a, buf_b, local, shared,
                    sem_a, sem_b):
    mrg = shared
    sid = lax.axis_index("s")
    wid = lax.axis_index("c") * NS + sid
    half = wid % 2
    prow = wid // 2
    pbase = (prow * PAIR_ROWS).astype(jnp.int32)
    ibase = half * (M // 2)
    iota = lax.iota(jnp.int32, L)
    zeros = jnp.zeros((L,), jnp.int32)
    limit = jnp.uint32(PAIR_ROWS)

    pltpu.async_copy(idx_hbm.at[pl.ds(sid * STG, STG)],
                     shared.at[pl.ds(sid * STG, STG)], sem_a)

    @pl.when(sid == 0)
    def _tail():
        pltpu.async_copy(idx_hbm.at[pl.ds(NS * STG, STG_TAIL)],
                         shared.at[pl.ds(NS * STG, STG_TAIL)], sem_b)

    @plsc.parallel_loop(0, LOCAL_PAD // L, 1, unroll=4)
    def _zero_body(j):
        local[pl.ds(j * L, L)] = zeros

    pltpu.make_async_copy(idx_hbm.at[pl.ds(0, STG)],
                          shared.at[pl.ds(0, STG)], sem_a).wait()

    @pl.when(sid == 0)
    def _tail_wait():
        pltpu.make_async_copy(idx_hbm.at[pl.ds(0, STG_TAIL)],
                              shared.at[pl.ds(0, STG_TAIL)], sem_b).wait()

    plsc.subcore_barrier()

    def _process(buf, chunk_base):
        @plsc.parallel_loop(0, VPC, 1, unroll=25)
        def _vec_body(v):
            ivec = buf[pl.ds(v * L, L)]
            val = (chunk_base + v * L) + iota
            loc = ivec - pbase
            m = plsc.bitcast(loc, jnp.uint32) < limit
            plsc.store_scatter(local, [loc], val, mask=m)

    pltpu.async_copy(shared.at[pl.ds(ibase, CHUNK)], buf_a, sem_a)

    def _pair_body(c, _):
        c0 = 2 * c
        pltpu.make_async_copy(shared.at[pl.ds(0, CHUNK)], buf_a, sem_a).wait()
        pltpu.async_copy(
            shared.at[pl.ds(ibase + (c0 + 1) * CHUNK, CHUNK)], buf_b, sem_b)
        _process(buf_a, (ibase + c0 * CHUNK).astype(jnp.int32))
        pltpu.make_async_copy(shared.at[pl.ds(0, CHUNK)], buf_b, sem_b).wait()
        pltpu.async_copy(
            shared.at[pl.ds(ibase + (c0 + 2) * CHUNK, CHUNK)], buf_a, sem_a)
        _process(buf_b, (ibase + (c0 + 1) * CHUNK).astype(jnp.int32))
        return 0

    lax.fori_loop(0, NCHALF // 2, _pair_body, 0)
    pltpu.make_async_copy(shared.at[pl.ds(0, CHUNK)], buf_a, sem_a).wait()
    _process(buf_a, (ibase + (NCHALF - 1) * CHUNK).astype(jnp.int32))

    plsc.subcore_barrier()

    @pl.when(half == 0)
    def _give_hi():
        pltpu.sync_copy(local.at[pl.ds(SPLIT, GIVE_HI)],
                        mrg.at[pl.ds(sid * SPLIT, GIVE_HI)])

    @pl.when(half == 1)
    def _give_lo():
        pltpu.sync_copy(local.at[pl.ds(0, SPLIT)],
                        mrg.at[pl.ds(sid * SPLIT, SPLIT)])

    plsc.subcore_barrier()

    def _merge(src_base, dst_base, mchunk, nchunk, unroll):
        bufs = (buf_a, buf_b)
        sems = (sem_a, sem_b)
        pltpu.async_copy(mrg.at[pl.ds(src_base, mchunk)],
                         bufs[0].at[pl.ds(0, mchunk)], sems[0])
        for k in range(nchunk):
            b, s = bufs[k % 2], sems[k % 2]
            pltpu.make_async_copy(mrg.at[pl.ds(0, mchunk)],
                                  b.at[pl.ds(0, mchunk)], s).wait()
            if k + 1 < nchunk:
                nb, ns_ = bufs[(k + 1) % 2], sems[(k + 1) % 2]
                pltpu.async_copy(
                    mrg.at[pl.ds(src_base + (k + 1) * mchunk, mchunk)],
                    nb.at[pl.ds(0, mchunk)], ns_)

            @plsc.parallel_loop(0, mchunk // L, 1, unroll=unroll)
            def _mx(j):
                o = dst_base + k * mchunk + j * L
                local[pl.ds(o, L)] = jnp.maximum(
                    local[pl.ds(o, L)], b[pl.ds(j * L, L)])

    @pl.when(half == 0)
    def _merge_lo():
        _merge((sid + 1) * SPLIT, 0, MRG_A, SPLIT // MRG_A, 8)
        pltpu.sync_copy(local.at[pl.ds(0, SPLIT)],
                        out_hbm.at[prow, pl.ds(0, SPLIT)])

    @pl.when(half == 1)
    def _merge_hi():
        _merge((sid - 1) * SPLIT, SPLIT, MRG_B, GIVE_HI // MRG_B, 5)
        pltpu.sync_copy(local.at[pl.ds(SPLIT, HI_ROWS)],
                        out_hbm.at[prow, pl.ds(SPLIT, HI_ROWS)])


def kernel(X, row_missing_idx):
    del X
    return _scatter_arange(row_missing_idx).reshape(-1)
